# Initial kernel scaffold; baseline (speedup 1.0000x reference)
import jax
import jax.numpy as jnp
from jax.experimental import pallas as pl


def kernel(x_tangent, adj_indices, adj_values):
    raise NotImplementedError("write your pallas kernel here")


# speedup vs baseline: 4.5177x; 4.5177x over previous
import functools

import jax
import jax.numpy as jnp
from jax import lax
from jax.experimental import pallas as pl
from jax.experimental.pallas import tpu as pltpu
from jax.experimental.pallas import tpu_sc as plsc

N_NODES = 10000
N_EDGES = 320000
D_FEAT = 128
C = 1.0
MIN_NORM = 1e-15
EPS = 1e-5

NUM_CORES = 2
NUM_SUBCORES = 16
NUM_TILES = NUM_CORES * NUM_SUBCORES
EDGES_PER_TILE = N_EDGES // NUM_TILES
CHUNK = 80
N_CHUNKS = EDGES_PER_TILE // CHUNK
ROWS_PER_TILE = 624
ROWS_TAIL = N_NODES - ROWS_PER_TILE * NUM_SUBCORES
LANES = 16
VPR = D_FEAT // LANES


def _sc_aggregate(x_tangent, row_idx, col_idx, vals, zeros_blk):
    mesh = plsc.VectorSubcoreMesh(
        core_axis_name="c", subcore_axis_name="s")

    @functools.partial(
        pl.kernel,
        out_type=jax.ShapeDtypeStruct((NUM_CORES, N_NODES, D_FEAT),
                                      jnp.float32),
        mesh=mesh,
        scratch_types=[
            pltpu.VMEM((CHUNK,), jnp.int32),
            pltpu.VMEM((CHUNK,), jnp.int32),
            pltpu.VMEM((CHUNK,), jnp.float32),
            pltpu.VMEM((CHUNK, D_FEAT), jnp.float32),
            pltpu.VMEM_SHARED((N_NODES, D_FEAT), jnp.float32),
            pltpu.SemaphoreType.DMA,
        ],
    )
    def agg(x_hbm, row_hbm, col_hbm, val_hbm, zero_hbm, out_hbm,
            colv, rowv, valv, buf, acc, sem):
        cid = lax.axis_index("c")
        sid = lax.axis_index("s")
        tid = sid * NUM_CORES + cid

        pltpu.sync_copy(zero_hbm, acc.at[pl.ds(sid * ROWS_PER_TILE,
                                               ROWS_PER_TILE)])

        @pl.when(sid == NUM_SUBCORES - 1)
        def _zero_tail():
            pltpu.sync_copy(
                zero_hbm.at[pl.ds(0, ROWS_TAIL)],
                acc.at[pl.ds(NUM_SUBCORES * ROWS_PER_TILE, ROWS_TAIL)])

        plsc.subcore_barrier()

        def chunk_body(i, carry):
            base = tid * EDGES_PER_TILE + i * CHUNK
            pltpu.sync_copy(col_hbm.at[pl.ds(base, CHUNK)], colv)
            pltpu.sync_copy(row_hbm.at[pl.ds(base, CHUNK)], rowv)
            pltpu.sync_copy(val_hbm.at[pl.ds(base, CHUNK)], valv)
            pltpu.async_copy(x_hbm.at[colv], buf, sem).wait()

            def scale(g, c2):
                vv = valv[pl.ds(g * LANES, LANES)]
                for j in range(LANES):
                    v = vv[j]
                    e = g * LANES + j
                    for k in range(VPR):
                        sl = pl.ds(k * LANES, LANES)
                        buf[e, sl] = buf[e, sl] * v
                return c2

            lax.fori_loop(0, CHUNK // LANES, scale, 0, unroll=False)
            pltpu.sync_copy(buf, acc.at[rowv], add=True)
            return carry

        lax.fori_loop(0, N_CHUNKS, chunk_body, 0, unroll=False)
        plsc.subcore_barrier()

        sl = pl.ds(sid * ROWS_PER_TILE, ROWS_PER_TILE)
        pltpu.sync_copy(acc.at[sl], out_hbm.at[cid, sl])

        @pl.when(sid == NUM_SUBCORES - 1)
        def _dump_tail():
            tl = pl.ds(NUM_SUBCORES * ROWS_PER_TILE, ROWS_TAIL)
            pltpu.sync_copy(acc.at[tl], out_hbm.at[cid, tl])

    return agg(x_tangent, row_idx, col_idx, vals, zeros_blk)


def _hyper_body(p_ref, o_ref):
    s = p_ref[0] + p_ref[1]
    sqrt_c = jnp.sqrt(C)
    nsq = jnp.sum(s * s, axis=-1, keepdims=True)
    u_norm = jnp.maximum(jnp.sqrt(nsq), MIN_NORM)
    gamma = jnp.tanh(sqrt_c * u_norm) * s / (sqrt_c * u_norm)
    gsq = jnp.sum(gamma * gamma, axis=-1, keepdims=True)
    g_norm = jnp.maximum(jnp.sqrt(gsq), MIN_NORM)
    maxnorm = (1.0 - EPS) / sqrt_c
    o_ref[...] = jnp.where(g_norm > maxnorm, gamma / g_norm * maxnorm, gamma)


def _hyper_project(partials):
    blk = 1000
    grid = N_NODES // blk
    return pl.pallas_call(
        _hyper_body,
        grid=(grid,),
        in_specs=[pl.BlockSpec((NUM_CORES, blk, D_FEAT),
                               lambda i: (0, i, 0))],
        out_specs=pl.BlockSpec((blk, D_FEAT), lambda i: (i, 0)),
        out_shape=jax.ShapeDtypeStruct((N_NODES, D_FEAT), jnp.float32),
    )(partials)


def kernel(x_tangent, adj_indices, adj_values):
    idx = adj_indices.astype(jnp.int32)
    row_idx = idx[0]
    col_idx = idx[1]
    zeros_blk = jnp.zeros((ROWS_PER_TILE, D_FEAT), jnp.float32)
    partials = _sc_aggregate(x_tangent, row_idx, col_idx, adj_values,
                             zeros_blk)
    return _hyper_project(partials)


# speedup vs baseline: 10.2625x; 2.2716x over previous
import functools

import jax
import jax.numpy as jnp
from jax import lax
from jax.experimental import pallas as pl
from jax.experimental.pallas import tpu as pltpu
from jax.experimental.pallas import tpu_sc as plsc

N_NODES = 10000
N_EDGES = 320000
D_FEAT = 128
C = 1.0
MIN_NORM = 1e-15
EPS = 1e-5

NUM_CORES = 2
NUM_SUBCORES = 16
NUM_TILES = NUM_CORES * NUM_SUBCORES
EDGES_PER_TILE = N_EDGES // NUM_TILES
CHUNK = 80
N_CHUNKS = EDGES_PER_TILE // CHUNK
ROWS_PER_TILE = 624
ROWS_TAIL = N_NODES - ROWS_PER_TILE * NUM_SUBCORES
LANES = 16
VPR = D_FEAT // LANES


def _sc_aggregate(x_tangent, row_idx, col_idx, vals, zeros_blk):
    mesh = plsc.VectorSubcoreMesh(
        core_axis_name="c", subcore_axis_name="s")

    @functools.partial(
        pl.kernel,
        out_type=jax.ShapeDtypeStruct((NUM_CORES, N_NODES, D_FEAT),
                                      jnp.float32),
        mesh=mesh,
        scratch_types=[
            pltpu.VMEM((EDGES_PER_TILE,), jnp.int32),
            pltpu.VMEM((2, CHUNK), jnp.int32),
            pltpu.VMEM((EDGES_PER_TILE,), jnp.float32),
            pltpu.VMEM((CHUNK, D_FEAT), jnp.float32),
            pltpu.VMEM((CHUNK, D_FEAT), jnp.float32),
            pltpu.VMEM_SHARED((N_NODES, D_FEAT), jnp.float32),
            pltpu.SemaphoreType.DMA,
            pltpu.SemaphoreType.DMA,
            pltpu.SemaphoreType.DMA,
            pltpu.SemaphoreType.DMA,
            pltpu.SemaphoreType.DMA,
        ],
    )
    def agg(x_hbm, row_hbm, col_hbm, val_hbm, zero_hbm, out_hbm,
            colv, rowv, valv, buf_a, buf_b, acc, sem_a, sem_b,
            rsem_a, rsem_b, isem):
        cid = lax.axis_index("c")
        sid = lax.axis_index("s")
        tid = sid * NUM_CORES + cid

        ebase = tid * EDGES_PER_TILE
        d1 = pltpu.async_copy(col_hbm.at[pl.ds(ebase, EDGES_PER_TILE)],
                              colv, isem)
        d3 = pltpu.async_copy(val_hbm.at[pl.ds(ebase, EDGES_PER_TILE)],
                              valv, isem)

        pltpu.sync_copy(zero_hbm, acc.at[pl.ds(sid * ROWS_PER_TILE,
                                               ROWS_PER_TILE)])

        @pl.when(sid == NUM_SUBCORES - 1)
        def _zero_tail():
            pltpu.sync_copy(
                zero_hbm.at[pl.ds(0, ROWS_TAIL)],
                acc.at[pl.ds(NUM_SUBCORES * ROWS_PER_TILE, ROWS_TAIL)])

        d1.wait()
        d3.wait()
        plsc.subcore_barrier()

        def row_start(i, par, rsem):
            pltpu.async_copy(row_hbm.at[pl.ds(ebase + i * CHUNK, CHUNK)],
                             rowv.at[par], rsem)

        def row_wait(i, par, rsem):
            pltpu.make_async_copy(
                row_hbm.at[pl.ds(ebase + i * CHUNK, CHUNK)],
                rowv.at[par], rsem).wait()

        def gather_start(i, buf, sem):
            pltpu.async_copy(x_hbm.at[colv.at[pl.ds(i * CHUNK, CHUNK)]],
                             buf, sem)

        def gather_wait(i, buf, sem):
            pltpu.make_async_copy(x_hbm.at[colv.at[pl.ds(i * CHUNK, CHUNK)]],
                                  buf, sem).wait()

        def scale(i, buf):
            def g(gi, c2):
                vv = valv[pl.ds(i * CHUNK + gi * LANES, LANES)]
                for j in range(LANES):
                    v = vv[j]
                    e = gi * LANES + j
                    for k in range(VPR):
                        sl = pl.ds(k * LANES, LANES)
                        buf[e, sl] = buf[e, sl] * v
                return c2

            lax.fori_loop(0, CHUNK // LANES, g, 0, unroll=False)

        def scatter(par, buf):
            pltpu.sync_copy(buf, acc.at[rowv.at[par]], add=True)

        gather_start(0, buf_a, sem_a)
        row_start(0, 0, rsem_a)

        def pair_body(p, carry):
            i0 = 2 * p
            i1 = 2 * p + 1
            gather_wait(i0, buf_a, sem_a)
            gather_start(i1, buf_b, sem_b)
            row_start(i1, 1, rsem_b)
            scale(i0, buf_a)
            row_wait(i0, 0, rsem_a)
            scatter(0, buf_a)
            gather_wait(i1, buf_b, sem_b)
            gather_start(i1 + 1, buf_a, sem_a)
            row_start(i1 + 1, 0, rsem_a)
            scale(i1, buf_b)
            row_wait(i1, 1, rsem_b)
            scatter(1, buf_b)
            return carry

        lax.fori_loop(0, (N_CHUNKS - 1) // 2, pair_body, 0, unroll=False)
        last = N_CHUNKS - 1
        gather_wait(last, buf_a, sem_a)
        scale(last, buf_a)
        row_wait(last, 0, rsem_a)
        scatter(0, buf_a)
        plsc.subcore_barrier()

        sl = pl.ds(sid * ROWS_PER_TILE, ROWS_PER_TILE)
        pltpu.sync_copy(acc.at[sl], out_hbm.at[cid, sl])

        @pl.when(sid == NUM_SUBCORES - 1)
        def _dump_tail():
            tl = pl.ds(NUM_SUBCORES * ROWS_PER_TILE, ROWS_TAIL)
            pltpu.sync_copy(acc.at[tl], out_hbm.at[cid, tl])

    return agg(x_tangent, row_idx, col_idx, vals, zeros_blk)


def _hyper_body(p_ref, o_ref):
    s = p_ref[0] + p_ref[1]
    sqrt_c = jnp.sqrt(C)
    nsq = jnp.sum(s * s, axis=-1, keepdims=True)
    u_norm = jnp.maximum(jnp.sqrt(nsq), MIN_NORM)
    gamma = jnp.tanh(sqrt_c * u_norm) * s / (sqrt_c * u_norm)
    gsq = jnp.sum(gamma * gamma, axis=-1, keepdims=True)
    g_norm = jnp.maximum(jnp.sqrt(gsq), MIN_NORM)
    maxnorm = (1.0 - EPS) / sqrt_c
    o_ref[...] = jnp.where(g_norm > maxnorm, gamma / g_norm * maxnorm, gamma)


def _hyper_project(partials):
    blk = 1000
    grid = N_NODES // blk
    return pl.pallas_call(
        _hyper_body,
        grid=(grid,),
        in_specs=[pl.BlockSpec((NUM_CORES, blk, D_FEAT),
                               lambda i: (0, i, 0))],
        out_specs=pl.BlockSpec((blk, D_FEAT), lambda i: (i, 0)),
        out_shape=jax.ShapeDtypeStruct((N_NODES, D_FEAT), jnp.float32),
    )(partials)


def kernel(x_tangent, adj_indices, adj_values):
    idx = adj_indices.astype(jnp.int32)
    row_idx = idx[0]
    col_idx = idx[1]
    vals = adj_values
    zeros_blk = jnp.zeros((ROWS_PER_TILE, D_FEAT), jnp.float32)
    partials = _sc_aggregate(x_tangent, row_idx, col_idx, vals,
                             zeros_blk)
    return _hyper_project(partials)


# speedup vs baseline: 10.6133x; 1.0342x over previous
import functools

import jax
import jax.numpy as jnp
from jax import lax
from jax.experimental import pallas as pl
from jax.experimental.pallas import tpu as pltpu
from jax.experimental.pallas import tpu_sc as plsc

N_NODES = 10000
N_EDGES = 320000
D_FEAT = 128
C = 1.0
MIN_NORM = 1e-15
EPS = 1e-5

NUM_CORES = 2
NUM_SUBCORES = 16
NUM_TILES = NUM_CORES * NUM_SUBCORES
EDGES_PER_TILE = N_EDGES // NUM_TILES
CHUNK = 80
N_CHUNKS = EDGES_PER_TILE // CHUNK
ROWS_PER_TILE = 624
ROWS_TAIL = N_NODES - ROWS_PER_TILE * NUM_SUBCORES
LANES = 16
VPR = D_FEAT // LANES


def _sc_aggregate(x_tangent, row_idx, col_idx, vals, zeros_blk):
    mesh = plsc.VectorSubcoreMesh(
        core_axis_name="c", subcore_axis_name="s")

    @functools.partial(
        pl.kernel,
        out_type=jax.ShapeDtypeStruct((NUM_CORES, N_NODES, D_FEAT),
                                      jnp.float32),
        mesh=mesh,
        scratch_types=[
            pltpu.VMEM((EDGES_PER_TILE,), jnp.int32),
            pltpu.VMEM((2, CHUNK), jnp.int32),
            pltpu.VMEM((EDGES_PER_TILE,), jnp.float32),
            pltpu.VMEM((CHUNK, D_FEAT), jnp.float32),
            pltpu.VMEM((CHUNK, D_FEAT), jnp.float32),
            pltpu.VMEM_SHARED((N_NODES, D_FEAT), jnp.float32),
            pltpu.SemaphoreType.DMA,
            pltpu.SemaphoreType.DMA,
            pltpu.SemaphoreType.DMA,
            pltpu.SemaphoreType.DMA,
            pltpu.SemaphoreType.DMA,
            pltpu.SemaphoreType.DMA,
            pltpu.SemaphoreType.DMA,
        ],
    )
    def agg(x_hbm, row_hbm, col_hbm, val_hbm, zero_hbm, out_hbm,
            colv, rowv, valv, buf_a, buf_b, acc, sem_a, sem_b,
            rsem_a, rsem_b, ssem_a, ssem_b, isem):
        cid = lax.axis_index("c")
        sid = lax.axis_index("s")
        tid = sid * NUM_CORES + cid

        ebase = tid * EDGES_PER_TILE
        d1 = pltpu.async_copy(col_hbm.at[pl.ds(ebase, EDGES_PER_TILE)],
                              colv, isem)
        d3 = pltpu.async_copy(val_hbm.at[pl.ds(ebase, EDGES_PER_TILE)],
                              valv, isem)

        pltpu.sync_copy(zero_hbm, acc.at[pl.ds(sid * ROWS_PER_TILE,
                                               ROWS_PER_TILE)])

        @pl.when(sid == NUM_SUBCORES - 1)
        def _zero_tail():
            pltpu.sync_copy(
                zero_hbm.at[pl.ds(0, ROWS_TAIL)],
                acc.at[pl.ds(NUM_SUBCORES * ROWS_PER_TILE, ROWS_TAIL)])

        d1.wait()
        d3.wait()
        plsc.subcore_barrier()

        def row_start(i, par, rsem):
            pltpu.async_copy(row_hbm.at[pl.ds(ebase + i * CHUNK, CHUNK)],
                             rowv.at[par], rsem)

        def row_wait(i, par, rsem):
            pltpu.make_async_copy(
                row_hbm.at[pl.ds(ebase + i * CHUNK, CHUNK)],
                rowv.at[par], rsem).wait()

        def gather_start(i, buf, sem):
            pltpu.async_copy(x_hbm.at[colv.at[pl.ds(i * CHUNK, CHUNK)]],
                             buf, sem)

        def gather_wait(i, buf, sem):
            pltpu.make_async_copy(x_hbm.at[colv.at[pl.ds(i * CHUNK, CHUNK)]],
                                  buf, sem).wait()

        def scale(i, buf):
            def g(gi, c2):
                vv = valv[pl.ds(i * CHUNK + gi * LANES, LANES)]
                for j in range(LANES):
                    v = vv[j]
                    e = gi * LANES + j
                    for k in range(VPR):
                        sl = pl.ds(k * LANES, LANES)
                        buf[e, sl] = buf[e, sl] * v
                return c2

            lax.fori_loop(0, CHUNK // LANES, g, 0, unroll=False)

        def scatter_start(par, buf, ssem):
            pltpu.async_copy(buf, acc.at[rowv.at[par]], ssem, add=True)

        def scatter_wait(par, buf, ssem):
            pltpu.make_async_copy(buf, acc.at[rowv.at[par]], ssem).wait()

        gather_start(0, buf_a, sem_a)
        row_start(0, 0, rsem_a)

        def pair_body(p, carry):
            i0 = 2 * p
            i1 = 2 * p + 1

            @pl.when(p > 0)
            def _drain_b():
                scatter_wait(1, buf_b, ssem_b)

            gather_start(i1, buf_b, sem_b)
            row_start(i1, 1, rsem_b)
            gather_wait(i0, buf_a, sem_a)
            scale(i0, buf_a)
            row_wait(i0, 0, rsem_a)
            scatter_start(0, buf_a, ssem_a)
            gather_wait(i1, buf_b, sem_b)
            scale(i1, buf_b)
            scatter_wait(0, buf_a, ssem_a)
            gather_start(i1 + 1, buf_a, sem_a)
            row_start(i1 + 1, 0, rsem_a)
            row_wait(i1, 1, rsem_b)
            scatter_start(1, buf_b, ssem_b)
            return carry

        lax.fori_loop(0, (N_CHUNKS - 1) // 2, pair_body, 0, unroll=False)
        last = N_CHUNKS - 1
        scatter_wait(1, buf_b, ssem_b)
        gather_wait(last, buf_a, sem_a)
        scale(last, buf_a)
        row_wait(last, 0, rsem_a)
        scatter_start(0, buf_a, ssem_a)
        scatter_wait(0, buf_a, ssem_a)
        plsc.subcore_barrier()

        sl = pl.ds(sid * ROWS_PER_TILE, ROWS_PER_TILE)
        pltpu.sync_copy(acc.at[sl], out_hbm.at[cid, sl])

        @pl.when(sid == NUM_SUBCORES - 1)
        def _dump_tail():
            tl = pl.ds(NUM_SUBCORES * ROWS_PER_TILE, ROWS_TAIL)
            pltpu.sync_copy(acc.at[tl], out_hbm.at[cid, tl])

    return agg(x_tangent, row_idx, col_idx, vals, zeros_blk)


def _hyper_body(p_ref, o_ref):
    s = p_ref[0] + p_ref[1]
    sqrt_c = jnp.sqrt(C)
    nsq = jnp.sum(s * s, axis=-1, keepdims=True)
    u_norm = jnp.maximum(jnp.sqrt(nsq), MIN_NORM)
    gamma = jnp.tanh(sqrt_c * u_norm) * s / (sqrt_c * u_norm)
    gsq = jnp.sum(gamma * gamma, axis=-1, keepdims=True)
    g_norm = jnp.maximum(jnp.sqrt(gsq), MIN_NORM)
    maxnorm = (1.0 - EPS) / sqrt_c
    o_ref[...] = jnp.where(g_norm > maxnorm, gamma / g_norm * maxnorm, gamma)


def _hyper_project(partials):
    blk = 1000
    grid = N_NODES // blk
    return pl.pallas_call(
        _hyper_body,
        grid=(grid,),
        in_specs=[pl.BlockSpec((NUM_CORES, blk, D_FEAT),
                               lambda i: (0, i, 0))],
        out_specs=pl.BlockSpec((blk, D_FEAT), lambda i: (i, 0)),
        out_shape=jax.ShapeDtypeStruct((N_NODES, D_FEAT), jnp.float32),
    )(partials)


def kernel(x_tangent, adj_indices, adj_values):
    idx = adj_indices.astype(jnp.int32)
    row_idx = idx[0]
    col_idx = idx[1]
    vals = adj_values
    zeros_blk = jnp.zeros((ROWS_PER_TILE, D_FEAT), jnp.float32)
    partials = _sc_aggregate(x_tangent, row_idx, col_idx, vals,
                             zeros_blk)
    return _hyper_project(partials)
